# baseline (speedup 1.0000x reference)
import functools
import math

import jax
import jax.numpy as jnp
from jax import lax
from jax.experimental import pallas as pl
from jax.experimental.pallas import tpu as pltpu
from jax.experimental.pallas import tpu_sc as plsc

LEVEL_HW = [(80, 80), (40, 40), (20, 20)]
SIZES = [h * w for h, w in LEVEL_HW]
OFFS = [0, SIZES[0], SIZES[0] + SIZES[1]]
TOTAL = sum(SIZES)
N = 2048
OUT_CH = 1024
FPN_CH = [256, 512, 1024]

_B1 = 1.0 / (1.0 + 2.0 * math.exp(-1.0))
_A1 = math.exp(-1.0) / (1.0 + 2.0 * math.exp(-1.0))

NC, NS = 2, 16
NW = NC * NS
BPW = N // NW


def _blur_ln_one(x, H, W):
    HW = H * W
    left = jnp.concatenate([x[:1], x[:-1]], axis=0)
    right = jnp.concatenate([x[1:], x[-1:]], axis=0)
    c_idx = lax.broadcasted_iota(jnp.int32, x.shape, 0) % W
    left = jnp.where(c_idx == 0, x, left)
    right = jnp.where(c_idx == W - 1, x, right)
    h = _B1 * x + _A1 * (left + right)
    up = jnp.concatenate([h[:W], h[:-W]], axis=0)
    dn = jnp.concatenate([h[W:], h[-W:]], axis=0)
    v = _B1 * h + _A1 * (up + dn)
    m = jnp.mean(v, axis=1, keepdims=True)
    var = jnp.mean((v - m) ** 2, axis=1, keepdims=True)
    return (v - m) / jnp.sqrt(var + 1e-5)


def _tables_body(p3_ref, p4_ref, p5_ref, t3_ref, t4_ref, t5_ref):
    for ref, out, (H, W) in ((p3_ref, t3_ref, LEVEL_HW[0]),
                             (p4_ref, t4_ref, LEVEL_HW[1]),
                             (p5_ref, t5_ref, LEVEL_HW[2])):
        x = ref[...].T
        out[...] = _blur_ln_one(x, H, W)


def _make_tables(p3f, p4f, p5f, interpret=False):
    out_shapes = tuple(
        jax.ShapeDtypeStruct((SIZES[i], FPN_CH[i]), jnp.float32)
        for i in range(3))
    return pl.pallas_call(
        _tables_body,
        out_shape=out_shapes,
        interpret=interpret,
    )(p3f, p4f, p5f)


def _fdiv(x, d):
    return ((x.astype(jnp.float32) + 0.5) * (1.0 / d)).astype(jnp.int32)


def _decode_lins(v):
    lvl1 = v >= OFFS[1]
    lvl2 = v >= OFFS[2]
    local = v - jnp.where(lvl2, OFFS[2], jnp.where(lvl1, OFFS[1], 0))
    r_src = jnp.where(lvl2, _fdiv(local, 20),
                      jnp.where(lvl1, _fdiv(local, 40), _fdiv(local, 80)))
    s_src = jnp.where(lvl2, 20, jnp.where(lvl1, 40, 80))
    c_src = local - r_src * s_src
    nc = 2 * c_src + 1
    nr = 2 * r_src + 1
    lins = []
    for S in (80, 40, 20):
        mc = nc * S
        mr = nr * S
        ct = jnp.where(lvl2, _fdiv(mc, 40),
                       jnp.where(lvl1, _fdiv(mc, 80), _fdiv(mc, 160)))
        rt = jnp.where(lvl2, _fdiv(mr, 40),
                       jnp.where(lvl1, _fdiv(mr, 80), _fdiv(mr, 160)))
        lins.append(rt * S + ct)
    return lins


def _gather_body(t3, t4, t5, fidx, g3, g4, g5,
                 idx_v, lin3, lin4, lin5, rows3, rows4, rows5, sem):
    wid = lax.axis_index("s") * NC + lax.axis_index("c")
    base = wid * BPW
    pltpu.sync_copy(fidx.at[pl.ds(base, BPW)], idx_v)
    for j in range(BPW // 16):
        sl = pl.ds(j * 16, 16)
        l0, l1, l2 = _decode_lins(idx_v[sl])
        lin3[sl] = l0
        lin4[sl] = l1
        lin5[sl] = l2
    cp3 = pltpu.async_copy(t3.at[lin3], rows3, sem)
    cp4 = pltpu.async_copy(t4.at[lin4], rows4, sem)
    cp5 = pltpu.async_copy(t5.at[lin5], rows5, sem)
    cp3.wait()
    cp4.wait()
    cp5.wait()
    pltpu.sync_copy(rows3, g3.at[pl.ds(base, BPW)])
    pltpu.sync_copy(rows4, g4.at[pl.ds(base, BPW)])
    pltpu.sync_copy(rows5, g5.at[pl.ds(base, BPW)])


def _gather_sc(t3, t4, t5, fidx, interpret=False):
    mesh = plsc.VectorSubcoreMesh(core_axis_name="c", subcore_axis_name="s",
                                  num_cores=NC, num_subcores=NS)
    out_type = tuple(
        jax.ShapeDtypeStruct((N, FPN_CH[i]), jnp.float32) for i in range(3))
    scratch = [
        pltpu.VMEM((BPW,), jnp.int32),
        pltpu.VMEM((BPW,), jnp.int32),
        pltpu.VMEM((BPW,), jnp.int32),
        pltpu.VMEM((BPW,), jnp.int32),
        pltpu.VMEM((BPW, FPN_CH[0]), jnp.float32),
        pltpu.VMEM((BPW, FPN_CH[1]), jnp.float32),
        pltpu.VMEM((BPW, FPN_CH[2]), jnp.float32),
        pltpu.SemaphoreType.DMA,
    ]
    k = pl.kernel(_gather_body, out_type=out_type, mesh=mesh,
                  scratch_types=scratch, interpret=interpret)
    return k(t3, t4, t5, fidx)


_ROWS_BLK = 256


def _head_body(g3, g4, g5, w3, w4, w5, wms, out):
    y3 = jnp.dot(g3[...].astype(jnp.bfloat16), w3[...],
                 preferred_element_type=jnp.float32)
    y4 = jnp.dot(g4[...].astype(jnp.bfloat16), w4[...],
                 preferred_element_type=jnp.float32)
    y5 = jnp.dot(g5[...].astype(jnp.bfloat16), w5[...],
                 preferred_element_type=jnp.float32)
    cat = jnp.concatenate([y3, y4, y5], axis=1)
    m = jnp.mean(cat, axis=1, keepdims=True)
    var = jnp.mean((cat - m) ** 2, axis=1, keepdims=True)
    ln = ((cat - m) / jnp.sqrt(var + 1e-5)).astype(jnp.bfloat16)
    out[...] = jnp.dot(ln, wms[...], preferred_element_type=jnp.float32)


def _head(g3, g4, g5, W3, W4, W5, Wms, interpret=False):
    nblk = N // _ROWS_BLK
    return pl.pallas_call(
        _head_body,
        grid=(nblk,),
        in_specs=[
            pl.BlockSpec((_ROWS_BLK, FPN_CH[0]), lambda i: (i, 0)),
            pl.BlockSpec((_ROWS_BLK, FPN_CH[1]), lambda i: (i, 0)),
            pl.BlockSpec((_ROWS_BLK, FPN_CH[2]), lambda i: (i, 0)),
            pl.BlockSpec((FPN_CH[0], OUT_CH), lambda i: (0, 0)),
            pl.BlockSpec((FPN_CH[1], OUT_CH), lambda i: (0, 0)),
            pl.BlockSpec((FPN_CH[2], OUT_CH), lambda i: (0, 0)),
            pl.BlockSpec((3 * OUT_CH, OUT_CH), lambda i: (0, 0)),
        ],
        out_specs=pl.BlockSpec((_ROWS_BLK, OUT_CH), lambda i: (i, 0)),
        out_shape=jax.ShapeDtypeStruct((N, OUT_CH), jnp.float32),
        interpret=interpret,
    )(g3, g4, g5, W3, W4, W5, Wms)


def kernel(p3, p4, p5, feat_idx, W3, W4, W5, Wms):
    p3f = p3[0].reshape(FPN_CH[0], SIZES[0])
    p4f = p4[0].reshape(FPN_CH[1], SIZES[1])
    p5f = p5[0].reshape(FPN_CH[2], SIZES[2])
    fidx = feat_idx.astype(jnp.int32)
    t3, t4, t5 = _make_tables(p3f, p4f, p5f)
    g3, g4, g5 = _gather_sc(t3, t4, t5, fidx)
    return _head(g3, g4, g5, W3.astype(jnp.bfloat16), W4.astype(jnp.bfloat16),
                 W5.astype(jnp.bfloat16), Wms.astype(jnp.bfloat16))


# speedup vs baseline: 1.3069x; 1.3069x over previous
import functools
import math

import jax
import jax.numpy as jnp
from jax import lax
from jax.experimental import pallas as pl
from jax.experimental.pallas import tpu as pltpu
from jax.experimental.pallas import tpu_sc as plsc

LEVEL_HW = [(80, 80), (40, 40), (20, 20)]
SIZES = [h * w for h, w in LEVEL_HW]
OFFS = [0, SIZES[0], SIZES[0] + SIZES[1]]
TOTAL = sum(SIZES)
N = 2048
OUT_CH = 1024
FPN_CH = [256, 512, 1024]

_B1 = 1.0 / (1.0 + 2.0 * math.exp(-1.0))
_A1 = math.exp(-1.0) / (1.0 + 2.0 * math.exp(-1.0))

NC, NS = 2, 16
NW = NC * NS
BPW = N // NW


def _pack_bf16_pair(a, b):
    ua = lax.bitcast_convert_type(a, jnp.uint32)
    ub = lax.bitcast_convert_type(b, jnp.uint32)
    ha = (ua + 0x7FFF + ((ua >> 16) & 1)) >> 16
    hb = (ub + 0x7FFF + ((ub >> 16) & 1)) >> 16
    return lax.bitcast_convert_type(ha | (hb << 16), jnp.int32)


_TGRID = 8


def _blur_rows(xfull, W, R):
    c_idx = lax.broadcasted_iota(jnp.int32, xfull.shape, 0) % W
    left = jnp.concatenate([xfull[:1], xfull[:-1]], axis=0)
    right = jnp.concatenate([xfull[1:], xfull[-1:]], axis=0)
    left = jnp.where(c_idx == 0, xfull, left)
    right = jnp.where(c_idx == W - 1, xfull, right)
    h = _B1 * xfull + _A1 * (left + right)
    return _B1 * h[W:W + R] + _A1 * (h[:R] + h[2 * W:2 * W + R])


def _blur_one(x, W):
    c_idx = lax.broadcasted_iota(jnp.int32, x.shape, 0) % W
    left = jnp.concatenate([x[:1], x[:-1]], axis=0)
    right = jnp.concatenate([x[1:], x[-1:]], axis=0)
    left = jnp.where(c_idx == 0, x, left)
    right = jnp.where(c_idx == W - 1, x, right)
    h = _B1 * x + _A1 * (left + right)
    up = jnp.concatenate([h[:W], h[:-W]], axis=0)
    dn = jnp.concatenate([h[W:], h[-W:]], axis=0)
    return _B1 * h + _A1 * (up + dn)


def _tables_body(x3, u3, d3, x4, u4, d4, x5, o3, o4, o5):
    for x, u, d, orr, (H, W), C in (
            (x3, u3, d3, o3, LEVEL_HW[0], FPN_CH[0]),
            (x4, u4, d4, o4, LEVEL_HW[1], FPN_CH[1])):
        R = (H * W) // _TGRID
        xfull = jnp.concatenate([u[...], x[...], d[...]], axis=0)
        b = _blur_rows(xfull, W, R)
        orr[...] = _pack_bf16_pair(b[:, :C // 2], b[:, C // 2:])

    @pl.when(pl.program_id(0) == 0)
    def _():
        b5 = _blur_one(x5[...], LEVEL_HW[2][1])
        o5[...] = _pack_bf16_pair(b5[:, :FPN_CH[2] // 2],
                                  b5[:, FPN_CH[2] // 2:])


def _mk_specs(lvl):
    H, W = LEVEL_HW[lvl]
    C = FPN_CH[lvl]
    R = (H * W) // _TGRID
    K = R // W
    nW = (H * W) // W

    def up_map(i):
        return (jnp.maximum(i * K - 1, 0), 0)

    def dn_map(i):
        return (jnp.minimum(i * K + K, nW - 1), 0)

    return [
        pl.BlockSpec((R, C), lambda i: (i, 0)),
        pl.BlockSpec((W, C), up_map),
        pl.BlockSpec((W, C), dn_map),
    ]


def _make_tables(p3f, p4f, p5f, interpret=False):
    in_specs = (_mk_specs(0) + _mk_specs(1)
                + [pl.BlockSpec((SIZES[2], FPN_CH[2]), lambda i: (0, 0))])
    out_specs = [
        pl.BlockSpec((SIZES[0] // _TGRID, FPN_CH[0] // 2), lambda i: (i, 0)),
        pl.BlockSpec((SIZES[1] // _TGRID, FPN_CH[1] // 2), lambda i: (i, 0)),
        pl.BlockSpec((SIZES[2], FPN_CH[2] // 2), lambda i: (0, 0)),
    ]
    return pl.pallas_call(
        _tables_body,
        grid=(_TGRID,),
        in_specs=in_specs,
        out_specs=out_specs,
        out_shape=tuple(
            jax.ShapeDtypeStruct((SIZES[i], FPN_CH[i] // 2), jnp.int32)
            for i in range(3)),
        interpret=interpret,
    )(p3f, p3f, p3f, p4f, p4f, p4f, p5f)


def _fdiv(x, d):
    return ((x.astype(jnp.float32) + 0.5) * (1.0 / d)).astype(jnp.int32)


def _decode_lins(v):
    lvl1 = v >= OFFS[1]
    lvl2 = v >= OFFS[2]
    local = v - jnp.where(lvl2, OFFS[2], jnp.where(lvl1, OFFS[1], 0))
    r_src = jnp.where(lvl2, _fdiv(local, 20),
                      jnp.where(lvl1, _fdiv(local, 40), _fdiv(local, 80)))
    s_src = jnp.where(lvl2, 20, jnp.where(lvl1, 40, 80))
    c_src = local - r_src * s_src
    nc = 2 * c_src + 1
    nr = 2 * r_src + 1
    lins = []
    for S in (80, 40, 20):
        mc = nc * S
        mr = nr * S
        ct = jnp.where(lvl2, _fdiv(mc, 40),
                       jnp.where(lvl1, _fdiv(mc, 80), _fdiv(mc, 160)))
        rt = jnp.where(lvl2, _fdiv(mr, 40),
                       jnp.where(lvl1, _fdiv(mr, 80), _fdiv(mr, 160)))
        lins.append(rt * S + ct)
    return lins


def _gather_body(t3, t4, t5, fidx, g3, g4, g5,
                 idx_v, lin3, lin4, lin5, rows3, rows4, rows5, sem):
    wid = lax.axis_index("s") * NC + lax.axis_index("c")
    base = wid * BPW
    pltpu.sync_copy(fidx.at[pl.ds(base, BPW)], idx_v)
    for j in range(BPW // 16):
        sl = pl.ds(j * 16, 16)
        l0, l1, l2 = _decode_lins(idx_v[sl])
        lin3[sl] = l0
        lin4[sl] = l1
        lin5[sl] = l2
    cp3 = pltpu.async_copy(t3.at[lin3], rows3, sem)
    cp4 = pltpu.async_copy(t4.at[lin4], rows4, sem)
    cp5 = pltpu.async_copy(t5.at[lin5], rows5, sem)
    cp3.wait()
    cp4.wait()
    cp5.wait()
    pltpu.sync_copy(rows3, g3.at[pl.ds(base, BPW)])
    pltpu.sync_copy(rows4, g4.at[pl.ds(base, BPW)])
    pltpu.sync_copy(rows5, g5.at[pl.ds(base, BPW)])


def _gather_sc(t3, t4, t5, fidx, interpret=False):
    mesh = plsc.VectorSubcoreMesh(core_axis_name="c", subcore_axis_name="s",
                                  num_cores=NC, num_subcores=NS)
    out_type = tuple(
        jax.ShapeDtypeStruct((N, FPN_CH[i] // 2), jnp.int32) for i in range(3))
    scratch = [
        pltpu.VMEM((BPW,), jnp.int32),
        pltpu.VMEM((BPW,), jnp.int32),
        pltpu.VMEM((BPW,), jnp.int32),
        pltpu.VMEM((BPW,), jnp.int32),
        pltpu.VMEM((BPW, FPN_CH[0] // 2), jnp.int32),
        pltpu.VMEM((BPW, FPN_CH[1] // 2), jnp.int32),
        pltpu.VMEM((BPW, FPN_CH[2] // 2), jnp.int32),
        pltpu.SemaphoreType.DMA,
    ]
    k = pl.kernel(_gather_body, out_type=out_type, mesh=mesh,
                  scratch_types=scratch, interpret=interpret)
    return k(t3, t4, t5, fidx)


_ROWS_BLK = 512


def _ln(x):
    m = jnp.mean(x, axis=1, keepdims=True)
    var = jnp.mean((x - m) ** 2, axis=1, keepdims=True)
    return (x - m) / jnp.sqrt(var + 1e-5)


def _unpack_bf16_pair(g):
    u = lax.bitcast_convert_type(g, jnp.uint32)
    lo = lax.bitcast_convert_type(u << 16, jnp.float32)
    hi = lax.bitcast_convert_type(u & jnp.uint32(0xFFFF0000), jnp.float32)
    return jnp.concatenate([lo, hi], axis=1)


def _head_body(g3, g4, g5, w3, w4, w5, wms, out):
    y3 = jnp.dot(_ln(_unpack_bf16_pair(g3[...])), w3[...],
                 preferred_element_type=jnp.float32)
    y4 = jnp.dot(_ln(_unpack_bf16_pair(g4[...])), w4[...],
                 preferred_element_type=jnp.float32)
    y5 = jnp.dot(_ln(_unpack_bf16_pair(g5[...])), w5[...],
                 preferred_element_type=jnp.float32)
    cat = jnp.concatenate([y3, y4, y5], axis=1)
    out[...] = jnp.dot(_ln(cat), wms[...], preferred_element_type=jnp.float32)


def _head(g3, g4, g5, W3, W4, W5, Wms, interpret=False):
    nblk = N // _ROWS_BLK
    return pl.pallas_call(
        _head_body,
        grid=(nblk,),
        in_specs=[
            pl.BlockSpec((_ROWS_BLK, FPN_CH[0] // 2), lambda i: (i, 0)),
            pl.BlockSpec((_ROWS_BLK, FPN_CH[1] // 2), lambda i: (i, 0)),
            pl.BlockSpec((_ROWS_BLK, FPN_CH[2] // 2), lambda i: (i, 0)),
            pl.BlockSpec((FPN_CH[0], OUT_CH), lambda i: (0, 0)),
            pl.BlockSpec((FPN_CH[1], OUT_CH), lambda i: (0, 0)),
            pl.BlockSpec((FPN_CH[2], OUT_CH), lambda i: (0, 0)),
            pl.BlockSpec((3 * OUT_CH, OUT_CH), lambda i: (0, 0)),
        ],
        out_specs=pl.BlockSpec((_ROWS_BLK, OUT_CH), lambda i: (i, 0)),
        out_shape=jax.ShapeDtypeStruct((N, OUT_CH), jnp.float32),
        interpret=interpret,
    )(g3, g4, g5, W3, W4, W5, Wms)


def kernel(p3, p4, p5, feat_idx, W3, W4, W5, Wms):
    p3f = p3[0].reshape(FPN_CH[0], SIZES[0]).T
    p4f = p4[0].reshape(FPN_CH[1], SIZES[1]).T
    p5f = p5[0].reshape(FPN_CH[2], SIZES[2]).T
    fidx = feat_idx.astype(jnp.int32)
    t3, t4, t5 = _make_tables(p3f, p4f, p5f)
    g3, g4, g5 = _gather_sc(t3, t4, t5, fidx)
    return _head(g3, g4, g5, W3, W4, W5, Wms)


# speedup vs baseline: 1.3312x; 1.0185x over previous
import functools
import math

import jax
import jax.numpy as jnp
from jax import lax
from jax.experimental import pallas as pl
from jax.experimental.pallas import tpu as pltpu
from jax.experimental.pallas import tpu_sc as plsc

LEVEL_HW = [(80, 80), (40, 40), (20, 20)]
SIZES = [h * w for h, w in LEVEL_HW]
OFFS = [0, SIZES[0], SIZES[0] + SIZES[1]]
TOTAL = sum(SIZES)
N = 2048
OUT_CH = 1024
FPN_CH = [256, 512, 1024]

_B1 = 1.0 / (1.0 + 2.0 * math.exp(-1.0))
_A1 = math.exp(-1.0) / (1.0 + 2.0 * math.exp(-1.0))

NC, NS = 2, 16
NW = NC * NS
BPW = N // NW


def _pack_bf16_pair(a, b):
    ua = lax.bitcast_convert_type(a, jnp.uint32)
    ub = lax.bitcast_convert_type(b, jnp.uint32)
    ha = (ua + 0x7FFF + ((ua >> 16) & 1)) >> 16
    hb = (ub + 0x7FFF + ((ub >> 16) & 1)) >> 16
    return lax.bitcast_convert_type(ha | (hb << 16), jnp.int32)


_TGRID = 4


def _blur_rows(xfull, W, R):
    c_idx = lax.broadcasted_iota(jnp.int32, xfull.shape, 0) % W
    left = jnp.concatenate([xfull[:1], xfull[:-1]], axis=0)
    right = jnp.concatenate([xfull[1:], xfull[-1:]], axis=0)
    left = jnp.where(c_idx == 0, xfull, left)
    right = jnp.where(c_idx == W - 1, xfull, right)
    h = _B1 * xfull + _A1 * (left + right)
    return _B1 * h[W:W + R] + _A1 * (h[:R] + h[2 * W:2 * W + R])


def _blur_one(x, W):
    c_idx = lax.broadcasted_iota(jnp.int32, x.shape, 0) % W
    left = jnp.concatenate([x[:1], x[:-1]], axis=0)
    right = jnp.concatenate([x[1:], x[-1:]], axis=0)
    left = jnp.where(c_idx == 0, x, left)
    right = jnp.where(c_idx == W - 1, x, right)
    h = _B1 * x + _A1 * (left + right)
    up = jnp.concatenate([h[:W], h[:-W]], axis=0)
    dn = jnp.concatenate([h[W:], h[-W:]], axis=0)
    return _B1 * h + _A1 * (up + dn)


def _tables_body(x3, u3, d3, x4, u4, d4, x5, o3, o4, o5):
    for x, u, d, orr, (H, W), C in (
            (x3, u3, d3, o3, LEVEL_HW[0], FPN_CH[0]),
            (x4, u4, d4, o4, LEVEL_HW[1], FPN_CH[1])):
        R = (H * W) // _TGRID
        xfull = jnp.concatenate([u[...], x[...], d[...]], axis=0)
        b = _blur_rows(xfull, W, R)
        orr[...] = _pack_bf16_pair(b[:, :C // 2], b[:, C // 2:])

    @pl.when(pl.program_id(0) == 0)
    def _():
        b5 = _blur_one(x5[...], LEVEL_HW[2][1])
        o5[...] = _pack_bf16_pair(b5[:, :FPN_CH[2] // 2],
                                  b5[:, FPN_CH[2] // 2:])


def _mk_specs(lvl):
    H, W = LEVEL_HW[lvl]
    C = FPN_CH[lvl]
    R = (H * W) // _TGRID
    K = R // W
    nW = (H * W) // W

    def up_map(i):
        return (jnp.maximum(i * K - 1, 0), 0)

    def dn_map(i):
        return (jnp.minimum(i * K + K, nW - 1), 0)

    return [
        pl.BlockSpec((R, C), lambda i: (i, 0)),
        pl.BlockSpec((W, C), up_map),
        pl.BlockSpec((W, C), dn_map),
    ]


def _make_tables(p3f, p4f, p5f, interpret=False):
    in_specs = (_mk_specs(0) + _mk_specs(1)
                + [pl.BlockSpec((SIZES[2], FPN_CH[2]), lambda i: (0, 0))])
    out_specs = [
        pl.BlockSpec((SIZES[0] // _TGRID, FPN_CH[0] // 2), lambda i: (i, 0)),
        pl.BlockSpec((SIZES[1] // _TGRID, FPN_CH[1] // 2), lambda i: (i, 0)),
        pl.BlockSpec((SIZES[2], FPN_CH[2] // 2), lambda i: (0, 0)),
    ]
    return pl.pallas_call(
        _tables_body,
        grid=(_TGRID,),
        in_specs=in_specs,
        out_specs=out_specs,
        out_shape=tuple(
            jax.ShapeDtypeStruct((SIZES[i], FPN_CH[i] // 2), jnp.int32)
            for i in range(3)),
        interpret=interpret,
    )(p3f, p3f, p3f, p4f, p4f, p4f, p5f)


def _fdiv(x, d):
    return ((x.astype(jnp.float32) + 0.5) * (1.0 / d)).astype(jnp.int32)


def _decode_lins(v):
    lvl1 = v >= OFFS[1]
    lvl2 = v >= OFFS[2]
    local = v - jnp.where(lvl2, OFFS[2], jnp.where(lvl1, OFFS[1], 0))
    r_src = jnp.where(lvl2, _fdiv(local, 20),
                      jnp.where(lvl1, _fdiv(local, 40), _fdiv(local, 80)))
    s_src = jnp.where(lvl2, 20, jnp.where(lvl1, 40, 80))
    c_src = local - r_src * s_src
    nc = 2 * c_src + 1
    nr = 2 * r_src + 1
    lins = []
    for S in (80, 40, 20):
        mc = nc * S
        mr = nr * S
        ct = jnp.where(lvl2, _fdiv(mc, 40),
                       jnp.where(lvl1, _fdiv(mc, 80), _fdiv(mc, 160)))
        rt = jnp.where(lvl2, _fdiv(mr, 40),
                       jnp.where(lvl1, _fdiv(mr, 80), _fdiv(mr, 160)))
        lins.append(rt * S + ct)
    return lins


def _gather_body(t3, t4, t5, fidx, g3, g4, g5,
                 idx_v, lin3, lin4, lin5, rows3, rows4, rows5, sem):
    wid = lax.axis_index("s") * NC + lax.axis_index("c")
    base = wid * BPW
    pltpu.sync_copy(fidx.at[pl.ds(base, BPW)], idx_v)
    for j in range(BPW // 16):
        sl = pl.ds(j * 16, 16)
        l0, l1, l2 = _decode_lins(idx_v[sl])
        lin3[sl] = l0
        lin4[sl] = l1
        lin5[sl] = l2
    cp3 = pltpu.async_copy(t3.at[lin3], rows3, sem)
    cp4 = pltpu.async_copy(t4.at[lin4], rows4, sem)
    cp5 = pltpu.async_copy(t5.at[lin5], rows5, sem)
    cp3.wait()
    cp4.wait()
    cp5.wait()
    pltpu.sync_copy(rows3, g3.at[pl.ds(base, BPW)])
    pltpu.sync_copy(rows4, g4.at[pl.ds(base, BPW)])
    pltpu.sync_copy(rows5, g5.at[pl.ds(base, BPW)])


def _gather_sc(t3, t4, t5, fidx, interpret=False):
    mesh = plsc.VectorSubcoreMesh(core_axis_name="c", subcore_axis_name="s",
                                  num_cores=NC, num_subcores=NS)
    out_type = tuple(
        jax.ShapeDtypeStruct((N, FPN_CH[i] // 2), jnp.int32) for i in range(3))
    scratch = [
        pltpu.VMEM((BPW,), jnp.int32),
        pltpu.VMEM((BPW,), jnp.int32),
        pltpu.VMEM((BPW,), jnp.int32),
        pltpu.VMEM((BPW,), jnp.int32),
        pltpu.VMEM((BPW, FPN_CH[0] // 2), jnp.int32),
        pltpu.VMEM((BPW, FPN_CH[1] // 2), jnp.int32),
        pltpu.VMEM((BPW, FPN_CH[2] // 2), jnp.int32),
        pltpu.SemaphoreType.DMA,
    ]
    k = pl.kernel(_gather_body, out_type=out_type, mesh=mesh,
                  scratch_types=scratch, interpret=interpret)
    return k(t3, t4, t5, fidx)


_ROWS_BLK = 512


def _ln(x):
    m = jnp.mean(x, axis=1, keepdims=True)
    var = jnp.mean((x - m) ** 2, axis=1, keepdims=True)
    return (x - m) / jnp.sqrt(var + 1e-5)


def _unpack_bf16_pair(g):
    u = lax.bitcast_convert_type(g, jnp.uint32)
    lo = lax.bitcast_convert_type(u << 16, jnp.float32)
    hi = lax.bitcast_convert_type(u & jnp.uint32(0xFFFF0000), jnp.float32)
    return jnp.concatenate([lo, hi], axis=1)


def _head_body(g3, g4, g5, w3, w4, w5, wms, out):
    y3 = jnp.dot(_ln(_unpack_bf16_pair(g3[...])), w3[...],
                 preferred_element_type=jnp.float32)
    y4 = jnp.dot(_ln(_unpack_bf16_pair(g4[...])), w4[...],
                 preferred_element_type=jnp.float32)
    y5 = jnp.dot(_ln(_unpack_bf16_pair(g5[...])), w5[...],
                 preferred_element_type=jnp.float32)
    cat = jnp.concatenate([y3, y4, y5], axis=1)
    out[...] = jnp.dot(_ln(cat), wms[...], preferred_element_type=jnp.float32)


def _head(g3, g4, g5, W3, W4, W5, Wms, interpret=False):
    nblk = N // _ROWS_BLK
    return pl.pallas_call(
        _head_body,
        grid=(nblk,),
        in_specs=[
            pl.BlockSpec((_ROWS_BLK, FPN_CH[0] // 2), lambda i: (i, 0)),
            pl.BlockSpec((_ROWS_BLK, FPN_CH[1] // 2), lambda i: (i, 0)),
            pl.BlockSpec((_ROWS_BLK, FPN_CH[2] // 2), lambda i: (i, 0)),
            pl.BlockSpec((FPN_CH[0], OUT_CH), lambda i: (0, 0)),
            pl.BlockSpec((FPN_CH[1], OUT_CH), lambda i: (0, 0)),
            pl.BlockSpec((FPN_CH[2], OUT_CH), lambda i: (0, 0)),
            pl.BlockSpec((3 * OUT_CH, OUT_CH), lambda i: (0, 0)),
        ],
        out_specs=pl.BlockSpec((_ROWS_BLK, OUT_CH), lambda i: (i, 0)),
        out_shape=jax.ShapeDtypeStruct((N, OUT_CH), jnp.float32),
        interpret=interpret,
    )(g3, g4, g5, W3, W4, W5, Wms)


def kernel(p3, p4, p5, feat_idx, W3, W4, W5, Wms):
    p3f = p3[0].reshape(FPN_CH[0], SIZES[0]).T
    p4f = p4[0].reshape(FPN_CH[1], SIZES[1]).T
    p5f = p5[0].reshape(FPN_CH[2], SIZES[2]).T
    fidx = feat_idx.astype(jnp.int32)
    t3, t4, t5 = _make_tables(p3f, p4f, p5f)
    g3, g4, g5 = _gather_sc(t3, t4, t5, fidx)
    return _head(g3, g4, g5, W3, W4, W5, Wms)
